# baseline (speedup 1.0000x reference)
import functools

import jax
import jax.numpy as jnp
from jax.experimental import pallas as pl
from jax.experimental.pallas import tpu as pltpu

_ANTIGEN_IDX = 2
_TEMPERATURE = 0.15
_N_PIPE = 2
_SPLIT = 2


def _fused_body(chain_ref, *refs):
    s_refs = refs[:_N_PIPE]
    out_ref = refs[_N_PIPE]
    acc_ref = refs[_N_PIPE + 1]
    step = pl.program_id(0)
    n_steps = pl.num_programs(0)
    bsz = chain_ref.shape[0]
    n_tok = chain_ref.shape[1]
    bpp = bsz // _N_PIPE
    chunk = n_tok // _SPLIT

    for p, s_ref in enumerate(s_refs):
        s = s_ref[0]
        row = p * bpp + step // _SPLIT
        c = step % _SPLIT
        chain_row = chain_ref[row, pl.ds(c * chunk, chunk)]
        m = (chain_row == _ANTIGEN_IDX).astype(jnp.float32).reshape(chunk, 1)
        tot = jnp.sum(s, axis=0)
        ag = jnp.sum(s * m, axis=0)
        partial = jnp.stack([tot, ag], axis=0)

        @pl.when(c == 0)
        def _init(row=row, partial=partial):
            acc_ref[row] = partial

        if _SPLIT > 1:
            @pl.when(c != 0)
            def _acc(row=row, partial=partial):
                acc_ref[row] += partial

    @pl.when(step == n_steps - 1)
    def _loss():
        pooled = acc_ref[...]
        mask_all = (chain_ref[...] == _ANTIGEN_IDX).astype(jnp.float32)
        ag_cnt = jnp.sum(mask_all, axis=1)
        ab_cnt = n_tok - ag_cnt

        tot_s = pooled[:, 0, :]
        ag_s = pooled[:, 1, :]
        ab_s = tot_s - ag_s

        ab_emb = ab_s / jnp.maximum(ab_cnt, 1.0)[:, None]
        ag_emb = ag_s / jnp.maximum(ag_cnt, 1.0)[:, None]

        ab_n = ab_emb / jnp.maximum(
            jnp.sqrt(jnp.sum(ab_emb * ab_emb, axis=1, keepdims=True)), 1e-12)
        ag_n = ag_emb / jnp.maximum(
            jnp.sqrt(jnp.sum(ag_emb * ag_emb, axis=1, keepdims=True)), 1e-12)

        sim = jax.lax.dot_general(
            ab_n, ag_n, (((1,), (1,)), ((), ())),
            preferred_element_type=jnp.float32,
            precision=jax.lax.Precision.HIGHEST,
        ) / _TEMPERATURE

        valid = ag_cnt > 0.0
        neg_inf = jnp.asarray(-jnp.inf, dtype=sim.dtype)
        sim_m = jnp.where(valid[None, :], sim, neg_inf)
        mx = jnp.max(sim_m, axis=1, keepdims=True)
        mx_safe = jnp.where(jnp.isfinite(mx), mx, 0.0)
        lse = jnp.log(
            jnp.sum(jnp.exp(sim_m - mx_safe), axis=1, keepdims=True)) + mx

        eye = (jax.lax.broadcasted_iota(jnp.int32, sim.shape, 0)
               == jax.lax.broadcasted_iota(jnp.int32, sim.shape, 1))
        logp = sim - lse
        diag = jnp.sum(jnp.where(eye, logp, 0.0), axis=1)

        n_valid = jnp.sum(valid.astype(jnp.float32))
        loss = -jnp.sum(jnp.where(valid, diag, 0.0)) / n_valid
        out_ref[...] = loss[None, None]


@functools.partial(jax.jit, static_argnames=("interpret",))
def kernel(s_i, chain_type, interpret=False):
    bsz, n_tok, dim = s_i.shape
    bpp = bsz // _N_PIPE
    n_steps = bpp * _SPLIT
    chunk = n_tok // _SPLIT

    def s_spec(p):
        return pl.BlockSpec(
            (1, chunk, dim),
            lambda b, p=p: (p * bpp + b // _SPLIT, b % _SPLIT, 0))

    loss = pl.pallas_call(
        _fused_body,
        grid=(n_steps,),
        in_specs=[pl.BlockSpec((bsz, n_tok), lambda b: (0, 0))]
        + [s_spec(p) for p in range(_N_PIPE)],
        out_specs=pl.BlockSpec((1, 1), lambda b: (0, 0)),
        out_shape=jax.ShapeDtypeStruct((1, 1), jnp.float32),
        scratch_shapes=[pltpu.VMEM((bsz, 2, dim), jnp.float32)],
        interpret=interpret,
    )(chain_type, *([s_i] * _N_PIPE))

    return loss[0, 0]


# speedup vs baseline: 1.0681x; 1.0681x over previous
import functools

import jax
import jax.numpy as jnp
from jax.experimental import pallas as pl
from jax.experimental.pallas import tpu as pltpu

_ANTIGEN_IDX = 2
_TEMPERATURE = 0.15
_N_PIPE = 2
_SPLIT = 1


def _fused_body(chain_ref, *refs):
    s_refs = refs[:_N_PIPE]
    out_ref = refs[_N_PIPE]
    acc_ref = refs[_N_PIPE + 1]
    step = pl.program_id(0)
    n_steps = pl.num_programs(0)
    bsz = chain_ref.shape[0]
    n_tok = chain_ref.shape[1]
    bpp = bsz // _N_PIPE
    chunk = n_tok // _SPLIT

    for p, s_ref in enumerate(s_refs):
        s = s_ref[0]
        row = p * bpp + step // _SPLIT
        c = step % _SPLIT
        chain_row = chain_ref[row, pl.ds(c * chunk, chunk)]
        m = (chain_row == _ANTIGEN_IDX).astype(jnp.float32).reshape(chunk, 1)
        tot = jnp.sum(s, axis=0)
        ag = jnp.sum(s * m, axis=0)
        partial = jnp.stack([tot, ag], axis=0)

        @pl.when(c == 0)
        def _init(row=row, partial=partial):
            acc_ref[row] = partial

        if _SPLIT > 1:
            @pl.when(c != 0)
            def _acc(row=row, partial=partial):
                acc_ref[row] += partial

    @pl.when(step == n_steps - 1)
    def _loss():
        pooled = acc_ref[...]
        mask_all = (chain_ref[...] == _ANTIGEN_IDX).astype(jnp.float32)
        ag_cnt = jnp.sum(mask_all, axis=1)
        ab_cnt = n_tok - ag_cnt

        tot_s = pooled[:, 0, :]
        ag_s = pooled[:, 1, :]
        ab_s = tot_s - ag_s

        ab_emb = ab_s / jnp.maximum(ab_cnt, 1.0)[:, None]
        ag_emb = ag_s / jnp.maximum(ag_cnt, 1.0)[:, None]

        ab_n = ab_emb / jnp.maximum(
            jnp.sqrt(jnp.sum(ab_emb * ab_emb, axis=1, keepdims=True)), 1e-12)
        ag_n = ag_emb / jnp.maximum(
            jnp.sqrt(jnp.sum(ag_emb * ag_emb, axis=1, keepdims=True)), 1e-12)

        sim = jax.lax.dot_general(
            ab_n, ag_n, (((1,), (1,)), ((), ())),
            preferred_element_type=jnp.float32,
            precision=jax.lax.Precision.HIGHEST,
        ) / _TEMPERATURE

        valid = ag_cnt > 0.0
        neg_inf = jnp.asarray(-jnp.inf, dtype=sim.dtype)
        sim_m = jnp.where(valid[None, :], sim, neg_inf)
        mx = jnp.max(sim_m, axis=1, keepdims=True)
        mx_safe = jnp.where(jnp.isfinite(mx), mx, 0.0)
        lse = jnp.log(
            jnp.sum(jnp.exp(sim_m - mx_safe), axis=1, keepdims=True)) + mx

        eye = (jax.lax.broadcasted_iota(jnp.int32, sim.shape, 0)
               == jax.lax.broadcasted_iota(jnp.int32, sim.shape, 1))
        logp = sim - lse
        diag = jnp.sum(jnp.where(eye, logp, 0.0), axis=1)

        n_valid = jnp.sum(valid.astype(jnp.float32))
        loss = -jnp.sum(jnp.where(valid, diag, 0.0)) / n_valid
        out_ref[...] = loss[None, None]


@functools.partial(jax.jit, static_argnames=("interpret",))
def kernel(s_i, chain_type, interpret=False):
    bsz, n_tok, dim = s_i.shape
    bpp = bsz // _N_PIPE
    n_steps = bpp * _SPLIT
    chunk = n_tok // _SPLIT

    def s_spec(p):
        return pl.BlockSpec(
            (1, chunk, dim),
            lambda b, p=p: (p * bpp + b // _SPLIT, b % _SPLIT, 0))

    loss = pl.pallas_call(
        _fused_body,
        grid=(n_steps,),
        in_specs=[pl.BlockSpec((bsz, n_tok), lambda b: (0, 0))]
        + [s_spec(p) for p in range(_N_PIPE)],
        out_specs=pl.BlockSpec((1, 1), lambda b: (0, 0)),
        out_shape=jax.ShapeDtypeStruct((1, 1), jnp.float32),
        scratch_shapes=[pltpu.VMEM((bsz, 2, dim), jnp.float32)],
        interpret=interpret,
    )(chain_type, *([s_i] * _N_PIPE))

    return loss[0, 0]
